# baseline (speedup 1.0000x reference)
import jax
import jax.numpy as jnp
from jax import lax
from jax.experimental import pallas as pl
from jax.experimental.pallas import tpu as pltpu
from jax.experimental.pallas import tpu_sc as plsc

_B, _F, _C, _N = 128, 512, 3, 66
_G = (_C * _N) // 2
_RC = 8
_NCHUNK = _B // _RC
_UNITS = _G * _NCHUNK
_NW = 32
_NBI = 4
_NBO = 4
_K = 52


def _pair_slab_body(x_hbm, o_hbm, *scr):
    ins = tuple(zip(scr[0:4], scr[8:12]))
    outs = tuple(zip(scr[4:8], scr[12:16]))
    wid = lax.axis_index("s") * 2 + lax.axis_index("c")
    base = 49 * wid + lax.min(wid, 16)
    n_units = 49 + (wid < 16).astype(jnp.int32)

    def unit_coords(k):
        u = base + k
        g = lax.shift_right_logical(u, 4)
        r0 = lax.bitwise_and(u, 15) * _RC
        return u, g, r0

    def in_copy(k, slot):
        _, g, r0 = unit_coords(k)
        buf, sem = ins[slot]
        return pltpu.make_async_copy(
            x_hbm.at[g, :, pl.ds(r0, _RC), :], buf, sem)

    def out_copy(k, slot):
        _, g, r0 = unit_coords(k)
        buf, sem = outs[slot]
        return pltpu.make_async_copy(
            buf, o_hbm.at[g, pl.ds(r0, _RC), :], sem)

    def compute(in_b, out_b):
        def row(r, carry):
            for c in range(_F // 16):
                sl = pl.ds(c * 16, 16)
                out_b[r, sl] = in_b[0, r, sl] + in_b[1, r, sl]
            return carry
        lax.fori_loop(0, _RC, row, 0)

    for k0 in range(_NBI - 1):
        in_copy(k0, k0).start()

    def quad(p, carry):
        for b in range(_NBI):
            k = p * _NBI + b
            valid = k < n_units
            kw = lax.max(k - _NBO, 0)

            @pl.when(k + _NBI - 1 < n_units)
            def _prefetch():
                in_copy(k + _NBI - 1, (b + _NBI - 1) % _NBI).start()

            @pl.when(valid)
            def _wait_in():
                in_copy(k, b).wait()

            @pl.when((k >= _NBO) & (k - _NBO < n_units))
            def _wait_out():
                out_copy(kw, b % _NBO).wait()

            @pl.when(valid)
            def _go():
                compute(ins[b][0], outs[b % _NBO][0])
                out_copy(k, b % _NBO).start()
        return carry

    lax.fori_loop(0, _K // _NBI, quad, 0)

    for m in range(_K - _NBO, _K):
        @pl.when(m < n_units)
        def _final_drain():
            out_copy(m, m % _NBO).wait()


_pair_slab = pl.kernel(
    _pair_slab_body,
    out_type=jax.ShapeDtypeStruct((_G, _B, _F), jnp.float32),
    mesh=plsc.VectorSubcoreMesh(core_axis_name="c", subcore_axis_name="s"),
    compiler_params=pltpu.CompilerParams(
        needs_layout_passes=False, skip_device_barrier=True),
    scratch_types=(
        [pltpu.VMEM((2, _RC, _F), jnp.float32) for _ in range(4)]
        + [pltpu.VMEM((_RC, _F), jnp.float32) for _ in range(4)]
        + [pltpu.SemaphoreType.DMA for _ in range(8)]
    ),
)


def kernel(x):
    xt = x.transpose(2, 3, 0, 1).reshape(_G, 2, _B, _F)
    out = _pair_slab(xt)
    return out.reshape(_C, _N // 2, _B, _F).transpose(2, 3, 0, 1)


# speedup vs baseline: 1.0313x; 1.0313x over previous
import jax
import jax.numpy as jnp
from jax import lax
from jax.experimental import pallas as pl
from jax.experimental.pallas import tpu as pltpu
from jax.experimental.pallas import tpu_sc as plsc

_B, _F, _C, _N = 128, 512, 3, 66
_G = (_C * _N) // 2
_RC = 8
_NCHUNK = _B // _RC
_UNITS = _G * _NCHUNK
_NW = 32
_NBI = 4
_NBO = 4
_K = 52


def _pair_slab_body(x_hbm, o_hbm, *scr):
    ins = tuple(zip(scr[0:4], scr[8:12]))
    outs = tuple(zip(scr[4:8], scr[12:16]))
    wid = lax.axis_index("s") * 2 + lax.axis_index("c")

    def unit_coords(k):
        u = wid + k * _NW
        g = lax.shift_right_logical(u, 4)
        r0 = lax.bitwise_and(u, 15) * _RC
        return u, g, r0

    def in_copy(k, slot):
        _, g, r0 = unit_coords(k)
        buf, sem = ins[slot]
        return pltpu.make_async_copy(
            x_hbm.at[g, :, pl.ds(r0, _RC), :], buf, sem)

    def out_copy(k, slot):
        _, g, r0 = unit_coords(k)
        buf, sem = outs[slot]
        return pltpu.make_async_copy(
            buf, o_hbm.at[g, pl.ds(r0, _RC), :], sem)

    def compute(in_b, out_b):
        def row(r, carry):
            for c in range(_F // 16):
                sl = pl.ds(c * 16, 16)
                out_b[r, sl] = in_b[0, r, sl] + in_b[1, r, sl]
            return carry
        lax.fori_loop(0, _RC, row, 0)

    for k0 in range(_NBI - 1):
        in_copy(k0, k0).start()

    def quad(p, carry):
        for b in range(_NBI):
            k = p * _NBI + b
            u = wid + k * _NW
            valid = u < _UNITS
            kw = lax.max(k - _NBO, 0)

            @pl.when(u + (_NBI - 1) * _NW < _UNITS)
            def _prefetch():
                in_copy(k + _NBI - 1, (b + _NBI - 1) % _NBI).start()

            @pl.when(valid)
            def _wait_in():
                in_copy(k, b).wait()

            @pl.when((k >= _NBO) & (u - _NBO * _NW < _UNITS))
            def _wait_out():
                out_copy(kw, b % _NBO).wait()

            @pl.when(valid)
            def _go():
                compute(ins[b][0], outs[b % _NBO][0])
                out_copy(k, b % _NBO).start()
        return carry

    lax.fori_loop(0, _K // _NBI, quad, 0)

    for m in range(_K - _NBO, _K):
        u_m = wid + m * _NW

        @pl.when(u_m < _UNITS)
        def _final_drain():
            out_copy(m, m % _NBO).wait()


_pair_slab = pl.kernel(
    _pair_slab_body,
    out_type=jax.ShapeDtypeStruct((_G, _B, _F), jnp.float32),
    mesh=plsc.VectorSubcoreMesh(core_axis_name="c", subcore_axis_name="s"),
    compiler_params=pltpu.CompilerParams(
        needs_layout_passes=False, skip_device_barrier=True),
    scratch_types=(
        [pltpu.VMEM((2, _RC, _F), jnp.float32) for _ in range(4)]
        + [pltpu.VMEM((_RC, _F), jnp.float32) for _ in range(4)]
        + [pltpu.SemaphoreType.DMA for _ in range(8)]
    ),
)


def kernel(x):
    xt = x.transpose(2, 3, 0, 1).reshape(_G, 2, _B, _F)
    out = _pair_slab(xt)
    return out.reshape(_C, _N // 2, _B, _F).transpose(2, 3, 0, 1)
